# Initial kernel scaffold; baseline (speedup 1.0000x reference)
import jax
import jax.numpy as jnp
from jax.experimental import pallas as pl


def kernel(x, edge_index, W1, b1, W2, b2):
    raise NotImplementedError("write your pallas kernel here")


# speedup vs baseline: 16.7229x; 16.7229x over previous
import jax
import jax.numpy as jnp
from jax import lax
from jax.experimental import pallas as pl
from jax.experimental.pallas import tpu as pltpu
from jax.experimental.pallas import tpu_sc as plsc

N = 10000
NP = 10240
E = 160000
ET = E + N
NT = 16
NCH = 168
B = 64
EPAD = NT * NCH * B
RPT = NP // NT
EH = NCH // 2
HALF = 128
DROW = NP // HALF


def _rsqrt16(v):
    i = plsc.bitcast(v, jnp.int32)
    i = jnp.int32(0x5F3759DF) - lax.shift_right_logical(i, 1)
    y = plsc.bitcast(i, jnp.float32)
    for _ in range(3):
        y = y * (1.5 - 0.5 * v * y * y)
    return y


def _sc_body(xh, ed3, out, xp,
             ed_v, deg_v, dinv_v, id_v, sidx0, didx0, sidx1, didx1,
             rows_a, rows_b, shared_deg, shared_agg, sem_a, sem_b):
    c = lax.axis_index("c")
    s = lax.axis_index("s")
    half = c * NP
    base = s * RPT

    zeros16 = jnp.zeros((16,), jnp.float32)
    ones16 = jnp.ones((16,), jnp.float32)

    def _zrow(i, _):
        rows_a[i // 8, pl.ds((i % 8) * 16, 16)] = zeros16
        return 0
    lax.fori_loop(0, B * 8, _zrow, 0)
    nz = DROW // NT
    pltpu.sync_copy(rows_a.at[pl.ds(0, nz)], shared_deg.at[pl.ds(s * nz, nz)])

    def _zdeg(i, _):
        deg_v[i // 8, pl.ds((i % 8) * 16, 16)] = zeros16
        return 0
    lax.fori_loop(0, DROW * 8, _zdeg, 0)

    def _hist(i, _):
        j = i // (B // 16)
        k = i % (B // 16)
        p = ed_v[j, pl.ds(k * 16, 16)]
        dv = p & 16383
        plsc.addupdate_scatter(
            deg_v, [lax.shift_right_logical(dv, 7), dv & 127], ones16)
        return 0

    for h in range(NCH // EH):
        pltpu.sync_copy(ed3.at[s * (NCH // EH) + h], ed_v)
        lax.fori_loop(0, EH * (B // 16), _hist, 0)

    for g in range(DROW // 16):
        id_v[pl.ds(g * 16, 16)] = lax.iota(jnp.int32, 16) + g * 16

    plsc.subcore_barrier()
    pltpu.sync_copy(deg_v, shared_deg.at[id_v], add=True)
    plsc.subcore_barrier()

    pltpu.sync_copy(shared_deg.at[pl.ds(s * nz, nz)], deg_v.at[pl.ds(0, nz)])

    def _dinv(i, _):
        v = deg_v[i // 8, pl.ds((i % 8) * 16, 16)]
        dinv_v[pl.ds(i * 16, 16)] = _rsqrt16(v)
        return 0
    lax.fori_loop(0, RPT // 16, _dinv, 0)

    for ch in range(RPT // B):
        pltpu.sync_copy(xh.at[pl.ds(half + base + ch * B, B)], rows_b)

        def _scale_in(r, _):
            idx16 = jnp.full((16,), ch * B + r, jnp.int32)
            d16 = plsc.load_gather(dinv_v, [idx16])
            for k in range(8):
                rows_b[r, pl.ds(k * 16, 16)] = rows_b[r, pl.ds(k * 16, 16)] * d16
            return 0
        lax.fori_loop(0, B, _scale_in, 0)
        pltpu.sync_copy(rows_b, xp.at[pl.ds(half + base + ch * B, B)])
        pltpu.sync_copy(rows_a, shared_agg.at[pl.ds(base + ch * B, B)])

    plsc.subcore_barrier()

    def _prep(j, sidx, didx):
        for g in range(B // 16):
            p = ed_v[j, pl.ds(g * 16, 16)]
            sidx[pl.ds(g * 16, 16)] = lax.shift_right_logical(p, 14) + half
            didx[pl.ds(g * 16, 16)] = p & 16383

    NI = EH // 2
    for h in range(NCH // EH):
        pltpu.sync_copy(ed3.at[s * (NCH // EH) + h], ed_v)
        _prep(0, sidx0, didx0)
        pltpu.async_copy(xp.at[sidx0], rows_a, sem_a)

        def _edges(i, _):
            _prep(2 * i + 1, sidx1, didx1)
            pltpu.make_async_copy(xp.at[sidx0], rows_a, sem_a).wait()
            pltpu.async_copy(xp.at[sidx1], rows_b, sem_b)
            pltpu.sync_copy(rows_a, shared_agg.at[didx0], add=True)

            @pl.when(i < NI - 1)
            def _():
                _prep(2 * i + 2, sidx0, didx0)
                pltpu.async_copy(xp.at[sidx0], rows_a, sem_a)

            pltpu.make_async_copy(xp.at[sidx1], rows_b, sem_b).wait()
            pltpu.sync_copy(rows_b, shared_agg.at[didx1], add=True)
            return 0
        lax.fori_loop(0, NI, _edges, 0)

    plsc.subcore_barrier()

    for ch in range(RPT // B):
        pltpu.sync_copy(shared_agg.at[pl.ds(base + ch * B, B)], rows_a)

        def _scale_out(r, _):
            idx16 = jnp.full((16,), ch * B + r, jnp.int32)
            d16 = plsc.load_gather(dinv_v, [idx16])
            for k in range(8):
                rows_a[r, pl.ds(k * 16, 16)] = rows_a[r, pl.ds(k * 16, 16)] * d16
            return 0
        lax.fori_loop(0, B, _scale_out, 0)
        pltpu.sync_copy(rows_a, out.at[pl.ds(half + base + ch * B, B)])


_sc_agg = pl.kernel(
    _sc_body,
    out_type=(
        jax.ShapeDtypeStruct((2 * NP, HALF), jnp.float32),
        jax.ShapeDtypeStruct((2 * NP, HALF), jnp.float32),
    ),
    mesh=plsc.VectorSubcoreMesh(core_axis_name="c", subcore_axis_name="s"),
    scratch_types=[
        pltpu.VMEM((EH, B), jnp.int32),
        pltpu.VMEM((DROW, HALF), jnp.float32),
        pltpu.VMEM((RPT,), jnp.float32),
        pltpu.VMEM((DROW,), jnp.int32),
        pltpu.VMEM((B,), jnp.int32),
        pltpu.VMEM((B,), jnp.int32),
        pltpu.VMEM((B,), jnp.int32),
        pltpu.VMEM((B,), jnp.int32),
        pltpu.VMEM((B, HALF), jnp.float32),
        pltpu.VMEM((B, HALF), jnp.float32),
        pltpu.VMEM_SHARED((DROW, HALF), jnp.float32),
        pltpu.VMEM_SHARED((NP, HALF), jnp.float32),
        pltpu.SemaphoreType.DMA,
        pltpu.SemaphoreType.DMA,
    ],
    compiler_params=pltpu.CompilerParams(needs_layout_passes=False),
)


def _mlp_body(x_ref, w1_ref, b1_ref, w2_ref, b2_ref, o_ref):
    h = jnp.dot(x_ref[...], w1_ref[...], preferred_element_type=jnp.float32)
    h = jnp.maximum(h + b1_ref[...], 0.0)
    o = jnp.dot(h, w2_ref[...], preferred_element_type=jnp.float32)
    o_ref[...] = jnp.maximum(o + b2_ref[...], 0.0)


_mlp = pl.pallas_call(
    _mlp_body,
    grid=(10,),
    in_specs=[
        pl.BlockSpec((1000, 256), lambda i: (i, 0)),
        pl.BlockSpec((256, 512), lambda i: (0, 0)),
        pl.BlockSpec((1, 512), lambda i: (0, 0)),
        pl.BlockSpec((512, 256), lambda i: (0, 0)),
        pl.BlockSpec((1, 256), lambda i: (0, 0)),
    ],
    out_specs=pl.BlockSpec((1000, 256), lambda i: (i, 0)),
    out_shape=jax.ShapeDtypeStruct((N, 256), jnp.float32),
)


def kernel(x, edge_index, W1, b1, W2, b2):
    x = x.astype(jnp.float32)
    ei = edge_index.astype(jnp.int32)
    loop = jnp.arange(N, dtype=jnp.int32)
    pad = EPAD - ET
    srcp = jnp.concatenate([ei[0], loop, jnp.zeros((pad,), jnp.int32)])
    dstp = jnp.concatenate([ei[1], loop, jnp.full((pad,), N, jnp.int32)])
    ed3 = (srcp * 16384 + dstp).reshape(NT * (NCH // EH), EH, B)
    xpad = jnp.zeros((NP, 256), jnp.float32).at[:N].set(x)
    xh = jnp.concatenate([xpad[:, :HALF], xpad[:, HALF:]], axis=0)

    agg2, _ = _sc_agg(xh, ed3)
    aggx = jnp.concatenate([agg2[:N], agg2[NP:NP + N]], axis=1)
    return _mlp(aggx, W1, b1.reshape(1, 512), W2, b2.reshape(1, 256))
